# Initial kernel scaffold; baseline (speedup 1.0000x reference)
import jax
import jax.numpy as jnp
from jax.experimental import pallas as pl


def kernel(image, audio, pos_image, pos_audio):
    raise NotImplementedError("write your pallas kernel here")


# speedup vs baseline: 1.1155x; 1.1155x over previous
import jax
import jax.numpy as jnp
from jax.experimental import pallas as pl

B, S, C = 4, 4096, 1024
BS = 256


def _pe_kernel(img_ref, aud_ref, pi_ref, pa_ref,
               oi_ref, oa_ref, pei_ref, pea_ref):
    pi = pi_ref[...]
    pa = pa_ref[...]
    pe_i = jnp.broadcast_to(pi[None], (B, BS, C))
    pe_a = jnp.broadcast_to(pa[None], (B, BS, C))
    oi_ref[...] = img_ref[...] + pe_i
    oa_ref[...] = aud_ref[...] + pe_a
    pei_ref[...] = pe_i
    pea_ref[...] = pe_a


def kernel(image, audio, pos_image, pos_audio):
    grid = (S // BS,)
    in_spec3 = pl.BlockSpec((B, BS, C), lambda s: (0, s, 0))
    in_spec2 = pl.BlockSpec((BS, C), lambda s: (s, 0))
    out_spec = pl.BlockSpec((B, BS, C), lambda s: (0, s, 0))
    out_shape = jax.ShapeDtypeStruct((B, S, C), jnp.float32)
    return pl.pallas_call(
        _pe_kernel,
        grid=grid,
        in_specs=[in_spec3, in_spec3, in_spec2, in_spec2],
        out_specs=[out_spec, out_spec, out_spec, out_spec],
        out_shape=[out_shape, out_shape, out_shape, out_shape],
    )(image, audio, pos_image, pos_audio)
